# baseline (speedup 1.0000x reference)
import functools

import jax
import jax.numpy as jnp
from jax import lax
from jax.experimental import pallas as pl
from jax.experimental.pallas import tpu as pltpu
from jax.experimental.pallas import tpu_sc as plsc

KP_EXTENT = 1.2
EXTENT = 0.6
NUM_KP = 15
KPAD = 16
H = 32
CIN = 128
COUT = 128
N = 10000
NPAD = 10240
BQ = 128
NBLK = NPAD // BQ
ROWS = BQ * H
D = CIN + 16
TOTAL_ROWS = NPAD * H
OD = 42
ODP = 48


def _sc_gather(table, sx, sy, sz, idx3, nw, chunks):
    per_w = chunks * 128
    total = nw * per_w
    mesh = plsc.VectorSubcoreMesh(core_axis_name="c", subcore_axis_name="s")
    nc = plsc.get_sparse_core_info().num_cores

    @functools.partial(
        pl.kernel,
        mesh=mesh,
        compiler_params=pltpu.CompilerParams(needs_layout_passes=False),
        out_type=(jax.ShapeDtypeStruct((total, CIN), jnp.float32),
                  jax.ShapeDtypeStruct((total * 16,), jnp.float32)),
        scratch_types=[
            pltpu.VMEM((chunks, 128), jnp.int32),
            pltpu.VMEM((128, CIN), jnp.float32),
            pltpu.VMEM((N,), jnp.float32),
            pltpu.VMEM((N,), jnp.float32),
            pltpu.VMEM((N,), jnp.float32),
            pltpu.VMEM((128 * 16,), jnp.float32),
            pltpu.SemaphoreType.DMA,
        ],
    )
    def gk(table_hbm, sx_hbm, sy_hbm, sz_hbm, idx_hbm, outf_hbm, outp_hbm,
           idx_v, rows_v, xt, yt, zt, pbuf, sem):
        wid = lax.axis_index("s") * nc + lax.axis_index("c")
        pltpu.sync_copy(idx_hbm.at[wid], idx_v)
        pltpu.sync_copy(sx_hbm, xt)
        pltpu.sync_copy(sy_hbm, yt)
        pltpu.sync_copy(sz_hbm, zt)
        base = wid * per_w

        def body(j, _):
            pltpu.async_copy(table_hbm.at[idx_v.at[j]], rows_v, sem).wait()
            off = pl.multiple_of(base + j * 128, 128)
            pltpu.sync_copy(rows_v, outf_hbm.at[pl.ds(off, 128)])
            for i in range(8):
                idx16 = idx_v[j, pl.ds(i * 16, 16)]
                pos = (lax.iota(jnp.int32, 16) + (i * 16)) * 16
                gx = plsc.load_gather(xt, [idx16])
                gy = plsc.load_gather(yt, [idx16])
                gz = plsc.load_gather(zt, [idx16])
                plsc.store_scatter(pbuf, [pos], gx)
                plsc.store_scatter(pbuf, [pos + 1], gy)
                plsc.store_scatter(pbuf, [pos + 2], gz)
            poff = pl.multiple_of((base + j * 128) * 16, 2048)
            pltpu.sync_copy(pbuf, outp_hbm.at[pl.ds(poff, 2048)])
            return _

        lax.fori_loop(0, chunks, body, None)

    return gk(table, sx, sy, sz, idx3)


def _wsum(w, nf):
    w = w.astype(jnp.bfloat16).astype(jnp.float32)
    nf = nf.astype(jnp.bfloat16).astype(jnp.float32)
    accs = [jnp.zeros((BQ, CIN), jnp.float32) for _ in range(NUM_KP)]
    for h in range(H):
        wh = w[h * BQ:(h + 1) * BQ, :]
        nfh = nf[h * BQ:(h + 1) * BQ, :]
        for k in range(NUM_KP):
            accs[k] = accs[k] + wh[:, k:k + 1] * nfh
    return jnp.concatenate(accs, axis=1)


def _rep_rows(x):
    return jnp.broadcast_to(x[None], (H, BQ, KPAD)).reshape(ROWS, KPAD)


def _tc_body(gf_ref, gx_ref, q_ref, kt_ref, wd_ref, db_ref, wgt_ref, o_ref):
    nf = gf_ref[...]
    xyz = gx_ref[...][:, 0:3]
    q = q_ref[...]
    qrep = jnp.broadcast_to(q[None], (H, BQ, 3)).reshape(ROWS, 3)
    npd = xyz - qrep
    np2 = jnp.sum(npd * npd, axis=1, keepdims=True)
    kt = kt_ref[...]

    npdot = (npd[:, 0:1] * kt[0:1, :] + npd[:, 1:2] * kt[1:2, :]
             + npd[:, 2:3] * kt[2:3, :])
    k2 = jnp.sum(kt * kt, axis=0, keepdims=True)
    sqd0 = jnp.maximum(np2 - 2.0 * npdot + k2, 0.0)
    w0 = jnp.maximum(1.0 - jnp.sqrt(sqd0) * (1.0 / EXTENT), 0.0)

    wf0 = _wsum(w0, nf)
    offm = jnp.dot(wf0.astype(jnp.bfloat16), wd_ref[...],
                   preferred_element_type=jnp.float32) + db_ref[...]

    dk = [kt[d:d + 1, :] + KP_EXTENT * offm[:, d * KPAD:(d + 1) * KPAD]
          for d in range(3)]
    dk2 = dk[0] * dk[0] + dk[1] * dk[1] + dk[2] * dk[2]
    npdot2 = (npd[:, 0:1] * _rep_rows(dk[0])
              + npd[:, 1:2] * _rep_rows(dk[1])
              + npd[:, 2:3] * _rep_rows(dk[2]))
    sqd = jnp.maximum(np2 - 2.0 * npdot2 + _rep_rows(dk2), 0.0)
    w1 = jnp.maximum(1.0 - jnp.sqrt(sqd) * (1.0 / EXTENT), 0.0)

    wf = _wsum(w1, nf)
    o_ref[...] = jnp.dot(wf.astype(jnp.bfloat16), wgt_ref[...],
                         preferred_element_type=jnp.float32)


def _tc_compute(gathered_f, gathered_x, qpad, ktpad, wd_perm, db_perm, w_flat):
    return pl.pallas_call(
        _tc_body,
        grid=(NBLK,),
        in_specs=[
            pl.BlockSpec((ROWS, CIN), lambda i: (i, 0)),
            pl.BlockSpec((ROWS, 16), lambda i: (i, 0)),
            pl.BlockSpec((BQ, 3), lambda i: (i, 0)),
            pl.BlockSpec((3, KPAD), lambda i: (0, 0)),
            pl.BlockSpec((NUM_KP * CIN, ODP), lambda i: (0, 0)),
            pl.BlockSpec((1, ODP), lambda i: (0, 0)),
            pl.BlockSpec((NUM_KP * CIN, COUT), lambda i: (0, 0)),
        ],
        out_specs=pl.BlockSpec((BQ, COUT), lambda i: (i, 0)),
        out_shape=jax.ShapeDtypeStruct((NPAD, COUT), jnp.float32),
    )(gathered_f, gathered_x, qpad, ktpad, wd_perm, db_perm, w_flat)


def kernel(query_points, support_points, neighbors, features, K_points,
           deformable_weight, dbias, weight):
    nb = jnp.pad(neighbors, ((0, NPAD - N), (0, 0)))
    idx = nb.reshape(NBLK, BQ, H).transpose(0, 2, 1).reshape(-1)

    info = plsc.get_sparse_core_info()
    nw = info.num_cores * info.num_subcores
    chunks = TOTAL_ROWS // (nw * 128)
    idx3 = idx.reshape(nw, chunks, 128)

    gathered_f, gathered_p = _sc_gather(
        features, support_points[:, 0], support_points[:, 1],
        support_points[:, 2], idx3, nw, chunks)
    gathered_x = gathered_p.reshape(TOTAL_ROWS, 16)

    qpad = jnp.pad(query_points, ((0, NPAD - N), (0, 0)))
    ktpad = jnp.pad(K_points, ((0, KPAD - NUM_KP), (0, 0))).T

    wd_flat = deformable_weight.reshape(NUM_KP * CIN, OD)
    cols = [j * 3 + d for d in range(3) for j in range(NUM_KP - 1)]
    wd_g = wd_flat[:, jnp.array(cols)].reshape(NUM_KP * CIN, 3, NUM_KP - 1)
    wd_perm = jnp.pad(wd_g, ((0, 0), (0, 0), (1, KPAD - NUM_KP))) \
        .reshape(NUM_KP * CIN, ODP).astype(jnp.bfloat16)
    db_g = dbias[jnp.array(cols)].reshape(1, 3, NUM_KP - 1)
    db_perm = jnp.pad(db_g, ((0, 0), (0, 0), (1, KPAD - NUM_KP))) \
        .reshape(1, ODP)

    w_flat = weight.reshape(NUM_KP * CIN, COUT).astype(jnp.bfloat16)

    out = _tc_compute(gathered_f, gathered_x, qpad, ktpad, wd_perm, db_perm,
                      w_flat)
    return out[:N]


# speedup vs baseline: 1.5662x; 1.5662x over previous
import functools

import jax
import jax.numpy as jnp
from jax import lax
from jax.experimental import pallas as pl
from jax.experimental.pallas import tpu as pltpu
from jax.experimental.pallas import tpu_sc as plsc

KP_EXTENT = 1.2
EXTENT = 0.6
NUM_KP = 15
KPAD = 16
H = 32
CIN = 128
COUT = 128
N = 10000
NPAD = 10240
BQ = 128
NBLK = NPAD // BQ
ROWS = BQ * H
D = CIN + 16
TOTAL_ROWS = NPAD * H
OD = 42
ODP = 48


def _sc_gather(table, sx, sy, sz, idx3, nw, chunks):
    per_w = chunks * 128
    total = nw * per_w
    mesh = plsc.VectorSubcoreMesh(core_axis_name="c", subcore_axis_name="s")
    nc = plsc.get_sparse_core_info().num_cores

    @functools.partial(
        pl.kernel,
        mesh=mesh,
        compiler_params=pltpu.CompilerParams(needs_layout_passes=False),
        out_type=(jax.ShapeDtypeStruct((total, CIN), jnp.float32),
                  jax.ShapeDtypeStruct((total * 16,), jnp.float32)),
        scratch_types=[
            pltpu.VMEM((chunks, 128), jnp.int32),
            pltpu.VMEM((128, CIN), jnp.float32),
            pltpu.VMEM((N,), jnp.float32),
            pltpu.VMEM((N,), jnp.float32),
            pltpu.VMEM((N,), jnp.float32),
            pltpu.VMEM((128 * 16,), jnp.float32),
            pltpu.SemaphoreType.DMA,
        ],
    )
    def gk(table_hbm, sx_hbm, sy_hbm, sz_hbm, idx_hbm, outf_hbm, outp_hbm,
           idx_v, rows_v, xt, yt, zt, pbuf, sem):
        wid = lax.axis_index("s") * nc + lax.axis_index("c")
        pltpu.sync_copy(idx_hbm.at[wid], idx_v)
        pltpu.sync_copy(sx_hbm, xt)
        pltpu.sync_copy(sy_hbm, yt)
        pltpu.sync_copy(sz_hbm, zt)
        base = wid * per_w

        def body(j, _):
            pltpu.async_copy(table_hbm.at[idx_v.at[j]], rows_v, sem).wait()
            off = pl.multiple_of(base + j * 128, 128)
            pltpu.sync_copy(rows_v, outf_hbm.at[pl.ds(off, 128)])
            for i in range(8):
                idx16 = idx_v[j, pl.ds(i * 16, 16)]
                pos = (lax.iota(jnp.int32, 16) + (i * 16)) * 16
                gx = plsc.load_gather(xt, [idx16])
                gy = plsc.load_gather(yt, [idx16])
                gz = plsc.load_gather(zt, [idx16])
                plsc.store_scatter(pbuf, [pos], gx)
                plsc.store_scatter(pbuf, [pos + 1], gy)
                plsc.store_scatter(pbuf, [pos + 2], gz)
            poff = pl.multiple_of((base + j * 128) * 16, 2048)
            pltpu.sync_copy(pbuf, outp_hbm.at[pl.ds(poff, 2048)])
            return _

        lax.fori_loop(0, chunks, body, None)

    return gk(table, sx, sy, sz, idx3)


def _wsum(w, nf):
    w3 = w.astype(jnp.bfloat16).reshape(BQ, H, KPAD)
    nf3 = nf.astype(jnp.bfloat16).reshape(BQ, H, CIN)
    return lax.dot_general(w3, nf3, (((1,), (1,)), ((0,), (0,))),
                           preferred_element_type=jnp.float32)


def _rep_rows(x):
    return jnp.broadcast_to(x[:, None, :], (BQ, H, KPAD)).reshape(ROWS, KPAD)


def _tc_body(gf_ref, gx_ref, q_ref, kt_ref, wd_ref, db_ref, wgt_ref, o_ref):
    nf = gf_ref[...]
    xyz = gx_ref[...][:, 0:3]
    q = q_ref[...]
    qrep = jnp.broadcast_to(q[:, None, :], (BQ, H, 3)).reshape(ROWS, 3)
    npd = xyz - qrep
    np2 = jnp.sum(npd * npd, axis=1, keepdims=True)
    kt = kt_ref[...]

    npdot = (npd[:, 0:1] * kt[0:1, :] + npd[:, 1:2] * kt[1:2, :]
             + npd[:, 2:3] * kt[2:3, :])
    k2 = jnp.sum(kt * kt, axis=0, keepdims=True)
    sqd0 = jnp.maximum(np2 - 2.0 * npdot + k2, 0.0)
    w0 = jnp.maximum(1.0 - jnp.sqrt(sqd0) * (1.0 / EXTENT), 0.0)

    wf0 = _wsum(w0, nf)
    offm = db_ref[...]
    for k in range(NUM_KP):
        offm = offm + jnp.dot(wf0[:, k, :].astype(jnp.bfloat16), wd_ref[k],
                              preferred_element_type=jnp.float32)

    dk = [kt[d:d + 1, :] + KP_EXTENT * offm[:, d * KPAD:(d + 1) * KPAD]
          for d in range(3)]
    dk2 = dk[0] * dk[0] + dk[1] * dk[1] + dk[2] * dk[2]
    npdot2 = (npd[:, 0:1] * _rep_rows(dk[0])
              + npd[:, 1:2] * _rep_rows(dk[1])
              + npd[:, 2:3] * _rep_rows(dk[2]))
    sqd = jnp.maximum(np2 - 2.0 * npdot2 + _rep_rows(dk2), 0.0)
    w1 = jnp.maximum(1.0 - jnp.sqrt(sqd) * (1.0 / EXTENT), 0.0)

    wf = _wsum(w1, nf)
    out = jnp.zeros((BQ, COUT), jnp.float32)
    for k in range(NUM_KP):
        out = out + jnp.dot(wf[:, k, :].astype(jnp.bfloat16), wgt_ref[k],
                            preferred_element_type=jnp.float32)
    o_ref[...] = out


def _tc_compute(gathered_f, gathered_x, qpad, ktpad, wd_perm, db_perm, w_flat):
    return pl.pallas_call(
        _tc_body,
        grid=(NBLK,),
        in_specs=[
            pl.BlockSpec((ROWS, CIN), lambda i: (i, 0)),
            pl.BlockSpec((ROWS, 16), lambda i: (i, 0)),
            pl.BlockSpec((BQ, 3), lambda i: (i, 0)),
            pl.BlockSpec((3, KPAD), lambda i: (0, 0)),
            pl.BlockSpec((NUM_KP, CIN, ODP), lambda i: (0, 0, 0)),
            pl.BlockSpec((1, ODP), lambda i: (0, 0)),
            pl.BlockSpec((NUM_KP, CIN, COUT), lambda i: (0, 0, 0)),
        ],
        out_specs=pl.BlockSpec((BQ, COUT), lambda i: (i, 0)),
        out_shape=jax.ShapeDtypeStruct((NPAD, COUT), jnp.float32),
    )(gathered_f, gathered_x, qpad, ktpad, wd_perm, db_perm, w_flat)


def kernel(query_points, support_points, neighbors, features, K_points,
           deformable_weight, dbias, weight):
    nb = jnp.pad(neighbors, ((0, NPAD - N), (0, 0)))
    idx = nb.reshape(-1)

    info = plsc.get_sparse_core_info()
    nw = info.num_cores * info.num_subcores
    chunks = TOTAL_ROWS // (nw * 128)
    idx3 = idx.reshape(nw, chunks, 128)

    gathered_f, gathered_p = _sc_gather(
        features, support_points[:, 0], support_points[:, 1],
        support_points[:, 2], idx3, nw, chunks)
    gathered_x = gathered_p.reshape(TOTAL_ROWS, 16)

    qpad = jnp.pad(query_points, ((0, NPAD - N), (0, 0)))
    ktpad = jnp.pad(K_points, ((0, KPAD - NUM_KP), (0, 0))).T

    wd_flat = deformable_weight.reshape(NUM_KP * CIN, OD)
    cols = [j * 3 + d for d in range(3) for j in range(NUM_KP - 1)]
    wd_g = wd_flat[:, jnp.array(cols)].reshape(NUM_KP * CIN, 3, NUM_KP - 1)
    wd_perm = jnp.pad(wd_g, ((0, 0), (0, 0), (1, KPAD - NUM_KP))) \
        .reshape(NUM_KP, CIN, ODP).astype(jnp.bfloat16)
    db_g = dbias[jnp.array(cols)].reshape(1, 3, NUM_KP - 1)
    db_perm = jnp.pad(db_g, ((0, 0), (0, 0), (1, KPAD - NUM_KP))) \
        .reshape(1, ODP)

    w_flat = weight.astype(jnp.bfloat16)

    out = _tc_compute(gathered_f, gathered_x, qpad, ktpad, wd_perm, db_perm,
                      w_flat)
    return out[:N]


# speedup vs baseline: 1.6764x; 1.0704x over previous
import functools

import jax
import jax.numpy as jnp
from jax import lax
from jax.experimental import pallas as pl
from jax.experimental.pallas import tpu as pltpu
from jax.experimental.pallas import tpu_sc as plsc

KP_EXTENT = 1.2
EXTENT = 0.6
NUM_KP = 15
KPAD = 16
H = 32
CIN = 128
COUT = 128
N = 10000
NPAD = 10240
BQ = 128
NBLK = NPAD // BQ
ROWS = BQ * H
D = CIN + 16
TOTAL_ROWS = NPAD * H
OD = 42
ODP = 48


def _sc_gather(table, sx, sy, sz, idx3, nw, chunks):
    per_w = chunks * 128
    total = nw * per_w
    GRP = 2
    GR = GRP * 128
    nround = chunks // (2 * GRP)
    mesh = plsc.VectorSubcoreMesh(core_axis_name="c", subcore_axis_name="s")
    nc = plsc.get_sparse_core_info().num_cores

    @functools.partial(
        pl.kernel,
        mesh=mesh,
        compiler_params=pltpu.CompilerParams(needs_layout_passes=False),
        out_type=(jax.ShapeDtypeStruct((total, CIN), jnp.float32),
                  jax.ShapeDtypeStruct((total * 16,), jnp.float32)),
        scratch_types=[
            pltpu.VMEM((chunks, 128), jnp.int32),
            pltpu.VMEM((GR, CIN), jnp.float32),
            pltpu.VMEM((GR, CIN), jnp.float32),
            pltpu.VMEM((N,), jnp.float32),
            pltpu.VMEM((N,), jnp.float32),
            pltpu.VMEM((N,), jnp.float32),
            pltpu.VMEM((GR * 16,), jnp.float32),
            pltpu.VMEM((GR * 16,), jnp.float32),
            pltpu.SemaphoreType.DMA,
            pltpu.SemaphoreType.DMA,
            pltpu.SemaphoreType.DMA,
            pltpu.SemaphoreType.DMA,
        ],
    )
    def gk(table_hbm, sx_hbm, sy_hbm, sz_hbm, idx_hbm, outf_hbm, outp_hbm,
           idx_v, rows0, rows1, xt, yt, zt, pbuf0, pbuf1,
           gsem0, gsem1, wsem0, wsem1):
        wid = lax.axis_index("s") * nc + lax.axis_index("c")
        pltpu.sync_copy(idx_hbm.at[wid], idx_v)
        pltpu.sync_copy(sx_hbm, xt)
        pltpu.sync_copy(sy_hbm, yt)
        pltpu.sync_copy(sz_hbm, zt)
        base = wid * per_w
        gsems = (gsem0, gsem1)
        wsems = (wsem0, wsem1)
        rows = (rows0, rows1)
        pbufs = (pbuf0, pbuf1)

        def wb_descs(b, g):
            off = pl.multiple_of(base + g * GR, GR)
            return (
                pltpu.make_async_copy(
                    rows[b], outf_hbm.at[pl.ds(off, GR)], wsems[b]),
                pltpu.make_async_copy(
                    pbufs[b], outp_hbm.at[pl.ds(off * 16, GR * 16)],
                    wsems[b]),
            )

        def g_desc(b, u, c):
            return pltpu.make_async_copy(
                table_hbm.at[idx_v.at[c]],
                rows[b].at[pl.ds(u * 128, 128)], gsems[b])

        def body(j, _):
            for b in range(2):
                g = j * 2 + b

                @pl.when(j > 0)
                def _():
                    for d in wb_descs(b, g - 2):
                        d.wait()

                for u in range(GRP):
                    g_desc(b, u, g * GRP + u).start()
            for b in range(2):
                g = j * 2 + b
                for u in range(GRP):
                    g_desc(b, u, g * GRP + u).wait()
                for i in range(GRP * 8):
                    c = g * GRP + i // 8
                    idx16 = idx_v[c, pl.ds((i % 8) * 16, 16)]
                    pos = (lax.iota(jnp.int32, 16) + (i * 16)) * 16
                    gx = plsc.load_gather(xt, [idx16])
                    gy = plsc.load_gather(yt, [idx16])
                    gz = plsc.load_gather(zt, [idx16])
                    pb = pbufs[b]
                    plsc.store_scatter(pb, [pos], gx)
                    plsc.store_scatter(pb, [pos + 1], gy)
                    plsc.store_scatter(pb, [pos + 2], gz)
                for d in wb_descs(b, g):
                    d.start()
            return _

        lax.fori_loop(0, nround, body, None)
        for b in range(2):
            for d in wb_descs(b, (nround - 1) * 2 + b):
                d.wait()

    return gk(table, sx, sy, sz, idx3)


def _wsum(w, nf):
    w3 = w.astype(jnp.bfloat16).reshape(BQ, H, KPAD)
    nf3 = nf.astype(jnp.bfloat16).reshape(BQ, H, CIN)
    return lax.dot_general(w3, nf3, (((1,), (1,)), ((0,), (0,))),
                           preferred_element_type=jnp.float32)


def _rep_rows(x):
    return jnp.broadcast_to(x[:, None, :], (BQ, H, KPAD)).reshape(ROWS, KPAD)


def _tc_body(gf_ref, gx_ref, q_ref, kt_ref, wd_ref, db_ref, wgt_ref, o_ref):
    nf = gf_ref[...]
    xyz = gx_ref[...][:, 0:3]
    q = q_ref[...]
    qrep = jnp.broadcast_to(q[:, None, :], (BQ, H, 3)).reshape(ROWS, 3)
    npd = xyz - qrep
    np2 = jnp.sum(npd * npd, axis=1, keepdims=True)
    kt = kt_ref[...]

    npdot = (npd[:, 0:1] * kt[0:1, :] + npd[:, 1:2] * kt[1:2, :]
             + npd[:, 2:3] * kt[2:3, :])
    k2 = jnp.sum(kt * kt, axis=0, keepdims=True)
    sqd0 = jnp.maximum(np2 - 2.0 * npdot + k2, 0.0)
    w0 = jnp.maximum(1.0 - jnp.sqrt(sqd0) * (1.0 / EXTENT), 0.0)

    wf0 = _wsum(w0, nf)
    offm = db_ref[...]
    for k in range(NUM_KP):
        offm = offm + jnp.dot(wf0[:, k, :].astype(jnp.bfloat16), wd_ref[k],
                              preferred_element_type=jnp.float32)

    dk = [kt[d:d + 1, :] + KP_EXTENT * offm[:, d * KPAD:(d + 1) * KPAD]
          for d in range(3)]
    dk2 = dk[0] * dk[0] + dk[1] * dk[1] + dk[2] * dk[2]
    npdot2 = (npd[:, 0:1] * _rep_rows(dk[0])
              + npd[:, 1:2] * _rep_rows(dk[1])
              + npd[:, 2:3] * _rep_rows(dk[2]))
    sqd = jnp.maximum(np2 - 2.0 * npdot2 + _rep_rows(dk2), 0.0)
    w1 = jnp.maximum(1.0 - jnp.sqrt(sqd) * (1.0 / EXTENT), 0.0)

    wf = _wsum(w1, nf)
    out = jnp.zeros((BQ, COUT), jnp.float32)
    for k in range(NUM_KP):
        out = out + jnp.dot(wf[:, k, :].astype(jnp.bfloat16), wgt_ref[k],
                            preferred_element_type=jnp.float32)
    o_ref[...] = out


def _tc_compute(gathered_f, gathered_x, qpad, ktpad, wd_perm, db_perm, w_flat):
    return pl.pallas_call(
        _tc_body,
        grid=(NBLK,),
        in_specs=[
            pl.BlockSpec((ROWS, CIN), lambda i: (i, 0)),
            pl.BlockSpec((ROWS, 16), lambda i: (i, 0)),
            pl.BlockSpec((BQ, 3), lambda i: (i, 0)),
            pl.BlockSpec((3, KPAD), lambda i: (0, 0)),
            pl.BlockSpec((NUM_KP, CIN, ODP), lambda i: (0, 0, 0)),
            pl.BlockSpec((1, ODP), lambda i: (0, 0)),
            pl.BlockSpec((NUM_KP, CIN, COUT), lambda i: (0, 0, 0)),
        ],
        out_specs=pl.BlockSpec((BQ, COUT), lambda i: (i, 0)),
        out_shape=jax.ShapeDtypeStruct((NPAD, COUT), jnp.float32),
    )(gathered_f, gathered_x, qpad, ktpad, wd_perm, db_perm, w_flat)


def kernel(query_points, support_points, neighbors, features, K_points,
           deformable_weight, dbias, weight):
    nb = jnp.pad(neighbors, ((0, NPAD - N), (0, 0)))
    idx = nb.reshape(-1)

    info = plsc.get_sparse_core_info()
    nw = info.num_cores * info.num_subcores
    chunks = TOTAL_ROWS // (nw * 128)
    idx3 = idx.reshape(nw, chunks, 128)

    gathered_f, gathered_p = _sc_gather(
        features, support_points[:, 0], support_points[:, 1],
        support_points[:, 2], idx3, nw, chunks)
    gathered_x = gathered_p.reshape(TOTAL_ROWS, 16)

    qpad = jnp.pad(query_points, ((0, NPAD - N), (0, 0)))
    ktpad = jnp.pad(K_points, ((0, KPAD - NUM_KP), (0, 0))).T

    wd_flat = deformable_weight.reshape(NUM_KP * CIN, OD)
    cols = [j * 3 + d for d in range(3) for j in range(NUM_KP - 1)]
    wd_g = wd_flat[:, jnp.array(cols)].reshape(NUM_KP * CIN, 3, NUM_KP - 1)
    wd_perm = jnp.pad(wd_g, ((0, 0), (0, 0), (1, KPAD - NUM_KP))) \
        .reshape(NUM_KP, CIN, ODP).astype(jnp.bfloat16)
    db_g = dbias[jnp.array(cols)].reshape(1, 3, NUM_KP - 1)
    db_perm = jnp.pad(db_g, ((0, 0), (0, 0), (1, KPAD - NUM_KP))) \
        .reshape(1, ODP)

    w_flat = weight.astype(jnp.bfloat16)

    out = _tc_compute(gathered_f, gathered_x, qpad, ktpad, wd_perm, db_perm,
                      w_flat)
    return out[:N]


# speedup vs baseline: 1.7715x; 1.0567x over previous
import functools

import jax
import jax.numpy as jnp
from jax import lax
from jax.experimental import pallas as pl
from jax.experimental.pallas import tpu as pltpu
from jax.experimental.pallas import tpu_sc as plsc

KP_EXTENT = 1.2
EXTENT = 0.6
NUM_KP = 15
KPAD = 16
H = 32
CIN = 128
COUT = 128
N = 10000
NPAD = 10240
BQ = 128
NBLK = NPAD // BQ
ROWS = BQ * H
D = CIN + 16
TOTAL_ROWS = NPAD * H
OD = 42
ODP = 48


def _sc_gather(table, sx, sy, sz, idx3, nw, chunks):
    per_w = chunks * 128
    total = nw * per_w
    GRP = 2
    GR = GRP * 128
    nround = chunks // (2 * GRP)
    mesh = plsc.VectorSubcoreMesh(core_axis_name="c", subcore_axis_name="s")
    nc = plsc.get_sparse_core_info().num_cores

    @functools.partial(
        pl.kernel,
        mesh=mesh,
        compiler_params=pltpu.CompilerParams(needs_layout_passes=False),
        out_type=(jax.ShapeDtypeStruct((total, CIN), jnp.float32),
                  jax.ShapeDtypeStruct((total * 16,), jnp.float32)),
        scratch_types=[
            pltpu.VMEM((chunks, 128), jnp.int32),
            pltpu.VMEM((GR, CIN), jnp.float32),
            pltpu.VMEM((GR, CIN), jnp.float32),
            pltpu.VMEM((N,), jnp.float32),
            pltpu.VMEM((N,), jnp.float32),
            pltpu.VMEM((N,), jnp.float32),
            pltpu.VMEM((GR * 16,), jnp.float32),
            pltpu.VMEM((GR * 16,), jnp.float32),
            pltpu.SemaphoreType.DMA,
            pltpu.SemaphoreType.DMA,
            pltpu.SemaphoreType.DMA,
            pltpu.SemaphoreType.DMA,
        ],
    )
    def gk(table_hbm, sx_hbm, sy_hbm, sz_hbm, idx_hbm, outf_hbm, outp_hbm,
           idx_v, rows0, rows1, xt, yt, zt, pbuf0, pbuf1,
           gsem0, gsem1, wsem0, wsem1):
        wid = lax.axis_index("s") * nc + lax.axis_index("c")
        pltpu.sync_copy(idx_hbm.at[wid], idx_v)
        pltpu.sync_copy(sx_hbm, xt)
        pltpu.sync_copy(sy_hbm, yt)
        pltpu.sync_copy(sz_hbm, zt)
        base = wid * per_w
        gsems = (gsem0, gsem1)
        wsems = (wsem0, wsem1)
        rows = (rows0, rows1)
        pbufs = (pbuf0, pbuf1)

        def wb_descs(b, g):
            off = pl.multiple_of(base + g * GR, GR)
            return (
                pltpu.make_async_copy(
                    rows[b], outf_hbm.at[pl.ds(off, GR)], wsems[b]),
                pltpu.make_async_copy(
                    pbufs[b], outp_hbm.at[pl.ds(off * 16, GR * 16)],
                    wsems[b]),
            )

        def g_desc(b, u, c):
            return pltpu.make_async_copy(
                table_hbm.at[idx_v.at[c]],
                rows[b].at[pl.ds(u * 128, 128)], gsems[b])

        def body(j, _):
            for b in range(2):
                g = j * 2 + b

                @pl.when(j > 0)
                def _():
                    for d in wb_descs(b, g - 2):
                        d.wait()

                for u in range(GRP):
                    g_desc(b, u, g * GRP + u).start()
            for b in range(2):
                g = j * 2 + b
                for i in range(GRP * 8):
                    c = g * GRP + i // 8
                    idx16 = idx_v[c, pl.ds((i % 8) * 16, 16)]
                    pos = (lax.iota(jnp.int32, 16) + (i * 16)) * 16
                    gx = plsc.load_gather(xt, [idx16])
                    gy = plsc.load_gather(yt, [idx16])
                    gz = plsc.load_gather(zt, [idx16])
                    pb = pbufs[b]
                    plsc.store_scatter(pb, [pos], gx)
                    plsc.store_scatter(pb, [pos + 1], gy)
                    plsc.store_scatter(pb, [pos + 2], gz)
                for u in range(GRP):
                    g_desc(b, u, g * GRP + u).wait()
                for d in wb_descs(b, g):
                    d.start()
            return _

        lax.fori_loop(0, nround, body, None)
        for b in range(2):
            for d in wb_descs(b, (nround - 1) * 2 + b):
                d.wait()

    return gk(table, sx, sy, sz, idx3)


def _wsum(w, nf):
    w3 = w.astype(jnp.bfloat16).reshape(BQ, H, KPAD)
    nf3 = nf.astype(jnp.bfloat16).reshape(BQ, H, CIN)
    return lax.dot_general(w3, nf3, (((1,), (1,)), ((0,), (0,))),
                           preferred_element_type=jnp.float32)


def _rep_rows(x):
    return jnp.broadcast_to(x[:, None, :], (BQ, H, KPAD)).reshape(ROWS, KPAD)


def _tc_body(gf_ref, gx_ref, q_ref, kt_ref, wd_ref, db_ref, wgt_ref, o_ref):
    nf = gf_ref[...]
    xyz = gx_ref[...][:, 0:3]
    q = q_ref[...]
    qrep = jnp.broadcast_to(q[:, None, :], (BQ, H, 3)).reshape(ROWS, 3)
    npd = xyz - qrep
    np2 = jnp.sum(npd * npd, axis=1, keepdims=True)
    kt = kt_ref[...]

    npdot = (npd[:, 0:1] * kt[0:1, :] + npd[:, 1:2] * kt[1:2, :]
             + npd[:, 2:3] * kt[2:3, :])
    k2 = jnp.sum(kt * kt, axis=0, keepdims=True)
    sqd0 = jnp.maximum(np2 - 2.0 * npdot + k2, 0.0)
    w0 = jnp.maximum(1.0 - jnp.sqrt(sqd0) * (1.0 / EXTENT), 0.0)

    wf0 = _wsum(w0, nf)
    offm = db_ref[...]
    for k in range(NUM_KP):
        offm = offm + jnp.dot(wf0[:, k, :].astype(jnp.bfloat16), wd_ref[k],
                              preferred_element_type=jnp.float32)

    dk = [kt[d:d + 1, :] + KP_EXTENT * offm[:, d * KPAD:(d + 1) * KPAD]
          for d in range(3)]
    dk2 = dk[0] * dk[0] + dk[1] * dk[1] + dk[2] * dk[2]
    npdot2 = (npd[:, 0:1] * _rep_rows(dk[0])
              + npd[:, 1:2] * _rep_rows(dk[1])
              + npd[:, 2:3] * _rep_rows(dk[2]))
    sqd = jnp.maximum(np2 - 2.0 * npdot2 + _rep_rows(dk2), 0.0)
    w1 = jnp.maximum(1.0 - jnp.sqrt(sqd) * (1.0 / EXTENT), 0.0)

    wf = _wsum(w1, nf)
    out = jnp.zeros((BQ, COUT), jnp.float32)
    for k in range(NUM_KP):
        out = out + jnp.dot(wf[:, k, :].astype(jnp.bfloat16), wgt_ref[k],
                            preferred_element_type=jnp.float32)
    o_ref[...] = out


def _tc_compute(gathered_f, gathered_x, qpad, ktpad, wd_perm, db_perm, w_flat):
    return pl.pallas_call(
        _tc_body,
        grid=(NBLK,),
        in_specs=[
            pl.BlockSpec((ROWS, CIN), lambda i: (i, 0)),
            pl.BlockSpec((ROWS, 16), lambda i: (i, 0)),
            pl.BlockSpec((BQ, 3), lambda i: (i, 0)),
            pl.BlockSpec((3, KPAD), lambda i: (0, 0)),
            pl.BlockSpec((NUM_KP, CIN, ODP), lambda i: (0, 0, 0)),
            pl.BlockSpec((1, ODP), lambda i: (0, 0)),
            pl.BlockSpec((NUM_KP, CIN, COUT), lambda i: (0, 0, 0)),
        ],
        out_specs=pl.BlockSpec((BQ, COUT), lambda i: (i, 0)),
        out_shape=jax.ShapeDtypeStruct((NPAD, COUT), jnp.float32),
    )(gathered_f, gathered_x, qpad, ktpad, wd_perm, db_perm, w_flat)


def kernel(query_points, support_points, neighbors, features, K_points,
           deformable_weight, dbias, weight):
    nb = jnp.pad(neighbors, ((0, NPAD - N), (0, 0)))
    idx = nb.reshape(-1)

    info = plsc.get_sparse_core_info()
    nw = info.num_cores * info.num_subcores
    chunks = TOTAL_ROWS // (nw * 128)
    idx3 = idx.reshape(nw, chunks, 128)

    gathered_f, gathered_p = _sc_gather(
        features, support_points[:, 0], support_points[:, 1],
        support_points[:, 2], idx3, nw, chunks)
    gathered_x = gathered_p.reshape(TOTAL_ROWS, 16)

    qpad = jnp.pad(query_points, ((0, NPAD - N), (0, 0)))
    ktpad = jnp.pad(K_points, ((0, KPAD - NUM_KP), (0, 0))).T

    wd_flat = deformable_weight.reshape(NUM_KP * CIN, OD)
    cols = [j * 3 + d for d in range(3) for j in range(NUM_KP - 1)]
    wd_g = wd_flat[:, jnp.array(cols)].reshape(NUM_KP * CIN, 3, NUM_KP - 1)
    wd_perm = jnp.pad(wd_g, ((0, 0), (0, 0), (1, KPAD - NUM_KP))) \
        .reshape(NUM_KP, CIN, ODP).astype(jnp.bfloat16)
    db_g = dbias[jnp.array(cols)].reshape(1, 3, NUM_KP - 1)
    db_perm = jnp.pad(db_g, ((0, 0), (0, 0), (1, KPAD - NUM_KP))) \
        .reshape(1, ODP)

    w_flat = weight.astype(jnp.bfloat16)

    out = _tc_compute(gathered_f, gathered_x, qpad, ktpad, wd_perm, db_perm,
                      w_flat)
    return out[:N]


# speedup vs baseline: 1.8141x; 1.0240x over previous
import functools

import jax
import jax.numpy as jnp
from jax import lax
from jax.experimental import pallas as pl
from jax.experimental.pallas import tpu as pltpu
from jax.experimental.pallas import tpu_sc as plsc

KP_EXTENT = 1.2
EXTENT = 0.6
NUM_KP = 15
KPAD = 16
H = 32
CIN = 128
COUT = 128
N = 10000
NPAD = 10240
BQ = 128
NBLK = NPAD // BQ
ROWS = BQ * H
D = CIN + 16
TOTAL_ROWS = NPAD * H
OD = 42
ODP = 48


def _sc_gather(table, sx, sy, sz, idx3, nw, chunks):
    per_w = chunks * 128
    total = nw * per_w
    GRP = 2
    GR = GRP * 128
    nround = chunks // (2 * GRP)
    mesh = plsc.VectorSubcoreMesh(core_axis_name="c", subcore_axis_name="s")
    nc = plsc.get_sparse_core_info().num_cores

    @functools.partial(
        pl.kernel,
        mesh=mesh,
        compiler_params=pltpu.CompilerParams(needs_layout_passes=False),
        out_type=(jax.ShapeDtypeStruct((total, CIN), jnp.float32),
                  jax.ShapeDtypeStruct((total * 16,), jnp.float32)),
        scratch_types=[
            pltpu.VMEM((chunks, 128), jnp.int32),
            pltpu.VMEM((GR, CIN), jnp.float32),
            pltpu.VMEM((GR, CIN), jnp.float32),
            pltpu.VMEM((N,), jnp.float32),
            pltpu.VMEM((N,), jnp.float32),
            pltpu.VMEM((N,), jnp.float32),
            pltpu.VMEM((GR * 16,), jnp.float32),
            pltpu.VMEM((GR * 16,), jnp.float32),
            pltpu.SemaphoreType.DMA,
            pltpu.SemaphoreType.DMA,
            pltpu.SemaphoreType.DMA,
            pltpu.SemaphoreType.DMA,
        ],
    )
    def gk(table_hbm, sx_hbm, sy_hbm, sz_hbm, idx_hbm, outf_hbm, outp_hbm,
           idx_v, rows0, rows1, xt, yt, zt, pbuf0, pbuf1,
           gsem0, gsem1, wsem0, wsem1):
        wid = lax.axis_index("s") * nc + lax.axis_index("c")
        pltpu.sync_copy(idx_hbm.at[wid], idx_v)
        pltpu.sync_copy(sx_hbm, xt)
        pltpu.sync_copy(sy_hbm, yt)
        pltpu.sync_copy(sz_hbm, zt)
        base = wid * per_w
        gsems = (gsem0, gsem1)
        wsems = (wsem0, wsem1)
        rows = (rows0, rows1)
        pbufs = (pbuf0, pbuf1)

        def wb_descs(b, g):
            off = pl.multiple_of(base + g * GR, GR)
            return (
                pltpu.make_async_copy(
                    rows[b], outf_hbm.at[pl.ds(off, GR)], wsems[b]),
                pltpu.make_async_copy(
                    pbufs[b], outp_hbm.at[pl.ds(off * 16, GR * 16)],
                    wsems[b]),
            )

        def g_desc(b, u, c):
            return pltpu.make_async_copy(
                table_hbm.at[idx_v.at[c]],
                rows[b].at[pl.ds(u * 128, 128)], gsems[b])

        def body(j, _):
            for b in range(2):
                g = j * 2 + b

                @pl.when(j > 0)
                def _():
                    for d in wb_descs(b, g - 2):
                        d.wait()

                for u in range(GRP):
                    g_desc(b, u, g * GRP + u).start()
            for b in range(2):
                g = j * 2 + b
                for i in range(GRP * 8):
                    c = g * GRP + i // 8
                    idx16 = idx_v[c, pl.ds((i % 8) * 16, 16)]
                    pos = (lax.iota(jnp.int32, 16) + (i * 16)) * 16
                    gx = plsc.load_gather(xt, [idx16])
                    gy = plsc.load_gather(yt, [idx16])
                    gz = plsc.load_gather(zt, [idx16])
                    pb = pbufs[b]
                    plsc.store_scatter(pb, [pos], gx)
                    plsc.store_scatter(pb, [pos + 1], gy)
                    plsc.store_scatter(pb, [pos + 2], gz)
                for u in range(GRP):
                    g_desc(b, u, g * GRP + u).wait()
                for d in wb_descs(b, g):
                    d.start()
            return _

        lax.fori_loop(0, nround, body, None)
        for b in range(2):
            for d in wb_descs(b, (nround - 1) * 2 + b):
                d.wait()

    return gk(table, sx, sy, sz, idx3)


def _wsum(w, nf):
    w3 = w.astype(jnp.bfloat16).reshape(BQ, H, KPAD)
    nf3 = nf.astype(jnp.bfloat16).reshape(BQ, H, CIN)
    return lax.dot_general(w3, nf3, (((1,), (1,)), ((0,), (0,))),
                           preferred_element_type=jnp.float32)


def _rep_rows(x):
    return jnp.broadcast_to(x[:, None, :], (BQ, H, KPAD)).reshape(ROWS, KPAD)


def _tc_body(gf_ref, gx_ref, q_ref, kt_ref, wd_ref, db_ref, wgt_ref, o_ref):
    nf = gf_ref[...]
    xyz = gx_ref[...][:, 0:3]
    q = q_ref[...]
    qrep = jnp.broadcast_to(q[:, None, :], (BQ, H, 3)).reshape(ROWS, 3)
    npd = xyz - qrep
    np2 = jnp.sum(npd * npd, axis=1, keepdims=True)
    kt = kt_ref[...]

    npdot = (npd[:, 0:1] * kt[0:1, :] + npd[:, 1:2] * kt[1:2, :]
             + npd[:, 2:3] * kt[2:3, :])
    k2 = jnp.sum(kt * kt, axis=0, keepdims=True)
    sqd0 = jnp.maximum(np2 - 2.0 * npdot + k2, 0.0)
    w0 = jnp.maximum(1.0 - jnp.sqrt(sqd0) * (1.0 / EXTENT), 0.0)

    wf0 = _wsum(w0, nf)
    offm = db_ref[...]
    for k in range(NUM_KP):
        offm = offm + jnp.dot(wf0[:, k, :].astype(jnp.bfloat16), wd_ref[k],
                              preferred_element_type=jnp.float32)

    dk = [kt[d:d + 1, :] + KP_EXTENT * offm[:, d * KPAD:(d + 1) * KPAD]
          for d in range(3)]
    dk2 = dk[0] * dk[0] + dk[1] * dk[1] + dk[2] * dk[2]
    npdot2 = (npd[:, 0:1] * _rep_rows(dk[0])
              + npd[:, 1:2] * _rep_rows(dk[1])
              + npd[:, 2:3] * _rep_rows(dk[2]))
    sqd = jnp.maximum(np2 - 2.0 * npdot2 + _rep_rows(dk2), 0.0)
    w1 = jnp.maximum(1.0 - jnp.sqrt(sqd) * (1.0 / EXTENT), 0.0)

    wf = _wsum(w1, nf)
    out = jnp.zeros((BQ, COUT), jnp.float32)
    for k in range(NUM_KP):
        out = out + jnp.dot(wf[:, k, :].astype(jnp.bfloat16), wgt_ref[k],
                            preferred_element_type=jnp.float32)
    o_ref[...] = out


def _tc_compute(gathered_f, gathered_x, qpad, ktpad, wd_perm, db_perm, w_flat):
    nblk = qpad.shape[0] // BQ
    return pl.pallas_call(
        _tc_body,
        grid=(nblk,),
        in_specs=[
            pl.BlockSpec((ROWS, CIN), lambda i: (i, 0)),
            pl.BlockSpec((ROWS, 16), lambda i: (i, 0)),
            pl.BlockSpec((BQ, 3), lambda i: (i, 0)),
            pl.BlockSpec((3, KPAD), lambda i: (0, 0)),
            pl.BlockSpec((NUM_KP, CIN, ODP), lambda i: (0, 0, 0)),
            pl.BlockSpec((1, ODP), lambda i: (0, 0)),
            pl.BlockSpec((NUM_KP, CIN, COUT), lambda i: (0, 0, 0)),
        ],
        out_specs=pl.BlockSpec((BQ, COUT), lambda i: (i, 0)),
        out_shape=jax.ShapeDtypeStruct((qpad.shape[0], COUT), jnp.float32),
    )(gathered_f, gathered_x, qpad, ktpad, wd_perm, db_perm, w_flat)


def kernel(query_points, support_points, neighbors, features, K_points,
           deformable_weight, dbias, weight):
    nb = jnp.pad(neighbors, ((0, NPAD - N), (0, 0)))
    idx = nb.reshape(-1)

    info = plsc.get_sparse_core_info()
    nw = info.num_cores * info.num_subcores
    nslice = 4
    srows = TOTAL_ROWS // nslice
    chunks = srows // (nw * 128)
    idx4 = idx.reshape(nslice, nw, chunks, 128)

    qpad = jnp.pad(query_points, ((0, NPAD - N), (0, 0)))
    ktpad = jnp.pad(K_points, ((0, KPAD - NUM_KP), (0, 0))).T

    wd_flat = deformable_weight.reshape(NUM_KP * CIN, OD)
    cols = [j * 3 + d for d in range(3) for j in range(NUM_KP - 1)]
    wd_g = wd_flat[:, jnp.array(cols)].reshape(NUM_KP * CIN, 3, NUM_KP - 1)
    wd_perm = jnp.pad(wd_g, ((0, 0), (0, 0), (1, KPAD - NUM_KP))) \
        .reshape(NUM_KP, CIN, ODP).astype(jnp.bfloat16)
    db_g = dbias[jnp.array(cols)].reshape(1, 3, NUM_KP - 1)
    db_perm = jnp.pad(db_g, ((0, 0), (0, 0), (1, KPAD - NUM_KP))) \
        .reshape(1, ODP)

    w_flat = weight.astype(jnp.bfloat16)

    spts = NPAD // nslice
    outs = []
    for sl in range(nslice):
        gf, gp = _sc_gather(
            features, support_points[:, 0], support_points[:, 1],
            support_points[:, 2], idx4[sl], nw, chunks)
        gx = gp.reshape(srows, 16)
        outs.append(_tc_compute(gf, gx, qpad[sl * spts:(sl + 1) * spts],
                                ktpad, wd_perm, db_perm, w_flat))
    out = jnp.concatenate(outs, axis=0)
    return out[:N]
